# Initial kernel scaffold; baseline (speedup 1.0000x reference)
import numpy as np
import jax
import jax.numpy as jnp
from jax.experimental import pallas as pl
from jax.experimental.pallas import tpu as pltpu

_B, _S, _H, _NH, _E, _TOPK = 2, 2048, 768, 12, 8, 2
_FF = 2 * _H
_DH = _H // _NH
_N = _B * _S
_BM = 256
_F = _S // 2 + 1
_FP = 2304
_KP = 2176

f32 = jnp.float32
bf16 = jnp.bfloat16

_LOCAL_E = (0, 2, 3, 5, 6)
_CONV_E = (0, 3, 6)
_FOUR_E = (1, 4, 7)


def _dft_consts():
    s = np.arange(_S)
    f = np.arange(_F)
    ang = 2.0 * np.pi * np.outer(f, s) / _S
    fwd = np.zeros((_FP, _S), np.float32)
    fwd[:_F] = np.cos(ang)
    fwd[_F:2 * _F] = -np.sin(ang)
    angT = ang.T
    gr = np.cos(angT) * (2.0 / _S)
    gr[:, 0] = 1.0 / _S
    gr[:, _F - 1] = np.cos(np.pi * s) / _S
    gi = -np.sin(angT) * (2.0 / _S)
    gi[:, 0] = 0.0
    gi[:, _F - 1] = 0.0
    inv = np.zeros((_S, _KP), np.float32)
    inv[:, :_F] = gr
    inv[:, _F:2 * _F] = gi
    return fwd, inv


_FWD_NP, _INV_NP = _dft_consts()


def _ln_mm_k(x_ref, g_ref, b_ref, w_ref, bb_ref, o_ref):
    x = x_ref[...]
    m = jnp.mean(x, -1, keepdims=True)
    xc = x - m
    v = jnp.mean(xc * xc, -1, keepdims=True)
    z = ((xc * jax.lax.rsqrt(v + 1e-5)) * g_ref[...] + b_ref[...]).astype(bf16)
    w = w_ref[...].astype(bf16)
    o_ref[...] = jnp.dot(z, w, preferred_element_type=f32) + bb_ref[...]


def _attn_k(q_ref, k_ref, v_ref, o_ref):
    q = q_ref[0].astype(bf16)
    k = k_ref[0].astype(bf16)
    s = jax.lax.dot_general(q, k, (((1,), (1,)), ((), ())),
                            preferred_element_type=f32) * (1.0 / np.sqrt(_DH))
    m = jnp.max(s, -1, keepdims=True)
    p = jnp.exp(s - m)
    den = jnp.sum(p, -1, keepdims=True)
    v = v_ref[0].astype(bf16)
    o = jnp.dot(p.astype(bf16), v, preferred_element_type=f32)
    o_ref[0] = o / den


def _proj_res_ln_k(o_ref, w_ref, bb_ref, xres_ref, g_ref, b_ref, x2_ref, y_ref):
    o = o_ref[...].astype(bf16)
    w = w_ref[...].astype(bf16)
    x2 = jnp.dot(o, w, preferred_element_type=f32) + bb_ref[...] + xres_ref[...]
    x2_ref[...] = x2
    m = jnp.mean(x2, -1, keepdims=True)
    xc = x2 - m
    v = jnp.mean(xc * xc, -1, keepdims=True)
    y_ref[...] = (xc * jax.lax.rsqrt(v + 1e-5)) * g_ref[...] + b_ref[...]


def _router_k(y_ref, wr_ref, br_ref, c_ref, aux_ref, me_ref, cnt_ref):
    i = pl.program_id(0)
    nblk = pl.num_programs(0)
    y = y_ref[...]
    logits = jnp.dot(y, wr_ref[...], preferred_element_type=f32) + br_ref[...]
    mx = jnp.max(logits, -1, keepdims=True)
    ex = jnp.exp(logits - mx)
    probs = ex / jnp.sum(ex, -1, keepdims=True)
    iota = jax.lax.broadcasted_iota(jnp.int32, probs.shape, 1)
    m1 = jnp.max(probs, -1, keepdims=True)
    i1 = jnp.min(jnp.where(probs == m1, iota, _E), -1, keepdims=True)
    p2 = jnp.where(iota == i1, -1.0, probs)
    m2 = jnp.max(p2, -1, keepdims=True)
    i2 = jnp.min(jnp.where(p2 == m2, iota, _E), -1, keepdims=True)
    den = m1 + m2
    sel1 = iota == i1
    sel2 = iota == i2
    c_ref[...] = (jnp.where(sel1, m1 / den, 0.0)
                  + jnp.where(sel2, m2 / den, 0.0))

    @pl.when(i == 0)
    def _():
        me_ref[...] = jnp.zeros_like(me_ref)
        cnt_ref[...] = jnp.zeros_like(cnt_ref)

    me_ref[...] += jnp.sum(probs, 0, keepdims=True)
    cnt_ref[...] += jnp.sum(jnp.where(sel1, 1.0, 0.0)
                            + jnp.where(sel2, 1.0, 0.0), 0, keepdims=True)

    @pl.when(i == nblk - 1)
    def _():
        me = me_ref[...] / _N
        ce = cnt_ref[...] / (_N * _TOPK)
        aux_ref[...] = (_E * jnp.sum(me * ce)).reshape(1, 1)


def _shared_k(y_ref, w1_ref, b1_ref, w2_ref, b2_ref, wg_ref, bg_ref, o_ref):
    y = y_ref[...]
    yb = y.astype(bf16)
    h = jnp.dot(yb, w1_ref[...].astype(bf16), preferred_element_type=f32) + b1_ref[...]
    h = jax.nn.gelu(h, approximate=False).astype(bf16)
    s = jnp.dot(h, w2_ref[...].astype(bf16), preferred_element_type=f32) + b2_ref[...]
    g = jnp.sum(y * wg_ref[...], -1, keepdims=True) + bg_ref[...]
    o_ref[...] = s * jax.nn.sigmoid(g)


def _convin_k(y_ref, k_ref, o_ref):
    y = y_ref[0]
    z = jnp.zeros((1, _H), f32)
    prev = jnp.concatenate([z, y[:-1]], 0)
    nxt = jnp.concatenate([y[1:], z], 0)
    for e in range(3):
        c = prev * k_ref[e, 0:1, :] + y * k_ref[e, 1:2, :] + nxt * k_ref[e, 2:3, :]
        o_ref[e, 0] = (y + c).astype(bf16)


def _emlp_k(x_ref, w1_ref, b1_ref, w2_ref, b2_ref, o_ref):
    x = x_ref[0]
    h = jnp.dot(x, w1_ref[0], preferred_element_type=f32) + b1_ref[0]
    h = jax.nn.gelu(h, approximate=False).astype(bf16)
    o_ref[0] = (jnp.dot(h, w2_ref[0], preferred_element_type=f32)
                + b2_ref[0]).astype(bf16)


def _mm_k(a_ref, b_ref, o_ref):
    o_ref[...] = jnp.dot(a_ref[...], b_ref[...], preferred_element_type=f32)


def _combine_k(x2_ref, base_ref, c_ref, el_ref, ef_ref, o_ref):
    out = x2_ref[...] + base_ref[...]
    c = c_ref[...]
    for j, e in enumerate(_LOCAL_E):
        out += c[:, e:e + 1] * el_ref[j].astype(f32)
    for j, e in enumerate(_FOUR_E):
        out += c[:, e:e + 1] * ef_ref[j].astype(f32)
    o_ref[...] = out


def _row(x):
    return x.reshape(1, -1)


def _ln_mm(x, g, b, w, bb):
    n, hi = x.shape
    ho = w.shape[1]
    return pl.pallas_call(
        _ln_mm_k,
        grid=(n // _BM,),
        in_specs=[pl.BlockSpec((_BM, hi), lambda i: (i, 0)),
                  pl.BlockSpec((1, hi), lambda i: (0, 0)),
                  pl.BlockSpec((1, hi), lambda i: (0, 0)),
                  pl.BlockSpec((hi, ho), lambda i: (0, 0)),
                  pl.BlockSpec((1, ho), lambda i: (0, 0))],
        out_specs=pl.BlockSpec((_BM, ho), lambda i: (i, 0)),
        out_shape=jax.ShapeDtypeStruct((n, ho), f32),
    )(x, _row(g), _row(b), w, _row(bb))


def _attention(q, k, v):
    gn = q.shape[0]
    bq = 512
    return pl.pallas_call(
        _attn_k,
        grid=(gn, _S // bq),
        in_specs=[pl.BlockSpec((1, bq, _DH), lambda h, i: (h, i, 0)),
                  pl.BlockSpec((1, _S, _DH), lambda h, i: (h, 0, 0)),
                  pl.BlockSpec((1, _S, _DH), lambda h, i: (h, 0, 0))],
        out_specs=pl.BlockSpec((1, bq, _DH), lambda h, i: (h, i, 0)),
        out_shape=jax.ShapeDtypeStruct((gn, _S, _DH), f32),
    )(q, k, v)


def _proj_res_ln(o, w, bb, xres, g, b):
    return pl.pallas_call(
        _proj_res_ln_k,
        grid=(_N // _BM,),
        in_specs=[pl.BlockSpec((_BM, _H), lambda i: (i, 0)),
                  pl.BlockSpec((_H, _H), lambda i: (0, 0)),
                  pl.BlockSpec((1, _H), lambda i: (0, 0)),
                  pl.BlockSpec((_BM, _H), lambda i: (i, 0)),
                  pl.BlockSpec((1, _H), lambda i: (0, 0)),
                  pl.BlockSpec((1, _H), lambda i: (0, 0))],
        out_specs=[pl.BlockSpec((_BM, _H), lambda i: (i, 0)),
                   pl.BlockSpec((_BM, _H), lambda i: (i, 0))],
        out_shape=[jax.ShapeDtypeStruct((_N, _H), f32),
                   jax.ShapeDtypeStruct((_N, _H), f32)],
    )(o, w, _row(bb), xres, _row(g), _row(b))


def _router(y, wr, br):
    c, aux = pl.pallas_call(
        _router_k,
        grid=(_N // _BM,),
        in_specs=[pl.BlockSpec((_BM, _H), lambda i: (i, 0)),
                  pl.BlockSpec((_H, _E), lambda i: (0, 0)),
                  pl.BlockSpec((1, _E), lambda i: (0, 0))],
        out_specs=[pl.BlockSpec((_BM, _E), lambda i: (i, 0)),
                   pl.BlockSpec((1, 1), lambda i: (0, 0))],
        out_shape=[jax.ShapeDtypeStruct((_N, _E), f32),
                   jax.ShapeDtypeStruct((1, 1), f32)],
        scratch_shapes=[pltpu.VMEM((1, _E), f32), pltpu.VMEM((1, _E), f32)],
    )(y, wr, _row(br))
    return c, aux


def _shared(y, mp):
    return pl.pallas_call(
        _shared_k,
        grid=(_N // _BM,),
        in_specs=[pl.BlockSpec((_BM, _H), lambda i: (i, 0)),
                  pl.BlockSpec((_H, _FF), lambda i: (0, 0)),
                  pl.BlockSpec((1, _FF), lambda i: (0, 0)),
                  pl.BlockSpec((_FF, _H), lambda i: (0, 0)),
                  pl.BlockSpec((1, _H), lambda i: (0, 0)),
                  pl.BlockSpec((1, _H), lambda i: (0, 0)),
                  pl.BlockSpec((1, 1), lambda i: (0, 0))],
        out_specs=pl.BlockSpec((_BM, _H), lambda i: (i, 0)),
        out_shape=jax.ShapeDtypeStruct((_N, _H), f32),
    )(y, mp['shared']['w1'], _row(mp['shared']['b1']),
      mp['shared']['w2'], _row(mp['shared']['b2']),
      mp['wg'].reshape(1, _H), mp['bg'].reshape(1, 1))


def _convin(y3, kstack):
    return pl.pallas_call(
        _convin_k,
        grid=(_B,),
        in_specs=[pl.BlockSpec((1, _S, _H), lambda b: (b, 0, 0)),
                  pl.BlockSpec((3, 3, _H), lambda b: (0, 0, 0))],
        out_specs=pl.BlockSpec((3, 1, _S, _H), lambda b: (0, b, 0, 0)),
        out_shape=jax.ShapeDtypeStruct((3, _B, _S, _H), bf16),
    )(y3, kstack)


def _experts_mlp(xs, w1, b1, w2, b2, bm, bcast_x=False):
    ne = w1.shape[0]
    n, din = xs.shape[1], xs.shape[2]
    dff = w1.shape[2]
    dout = w2.shape[2]
    if bcast_x:
        x_spec = pl.BlockSpec((1, bm, din), lambda e, i: (0, i, 0))
    else:
        x_spec = pl.BlockSpec((1, bm, din), lambda e, i: (e, i, 0))
    return pl.pallas_call(
        _emlp_k,
        grid=(ne, n // bm),
        in_specs=[x_spec,
                  pl.BlockSpec((1, din, dff), lambda e, i: (e, 0, 0)),
                  pl.BlockSpec((1, 1, dff), lambda e, i: (e, 0, 0)),
                  pl.BlockSpec((1, dff, dout), lambda e, i: (e, 0, 0)),
                  pl.BlockSpec((1, 1, dout), lambda e, i: (e, 0, 0))],
        out_specs=pl.BlockSpec((1, bm, dout), lambda e, i: (e, i, 0)),
        out_shape=jax.ShapeDtypeStruct((ne, n, dout), bf16),
    )(xs, w1, b1, w2, b2)


def _mm(a, b, bm_a, bn_b):
    ma, ka = a.shape
    kb, nb = b.shape
    return pl.pallas_call(
        _mm_k,
        grid=(ma // bm_a, nb // bn_b),
        in_specs=[pl.BlockSpec((bm_a, ka), lambda i, j: (i, 0)),
                  pl.BlockSpec((kb, bn_b), lambda i, j: (0, j))],
        out_specs=pl.BlockSpec((bm_a, bn_b), lambda i, j: (i, j)),
        out_shape=jax.ShapeDtypeStruct((ma, nb), f32),
    )(a, b)


def _combine(x2, base, c, el, ef):
    return pl.pallas_call(
        _combine_k,
        grid=(_N // _BM,),
        in_specs=[pl.BlockSpec((_BM, _H), lambda i: (i, 0)),
                  pl.BlockSpec((_BM, _H), lambda i: (i, 0)),
                  pl.BlockSpec((_BM, _E), lambda i: (i, 0)),
                  pl.BlockSpec((5, _BM, _H), lambda i: (0, i, 0)),
                  pl.BlockSpec((3, _BM, _H), lambda i: (0, i, 0))],
        out_specs=pl.BlockSpec((_BM, _H), lambda i: (i, 0)),
        out_shape=jax.ShapeDtypeStruct((_N, _H), f32),
    )(x2, base, c, el, ef)


def _layer(h, lp, fwd_c, inv_c):
    ap = lp['attn']
    mp = lp['moe']
    wqkv = jnp.concatenate([ap['wq'], ap['wk'], ap['wv']], axis=1)
    bqkv = jnp.concatenate([ap['bq'], ap['bk'], ap['bv']])
    qkv = _ln_mm(h, lp['ln1_g'], lp['ln1_b'], wqkv, bqkv)
    qkv = qkv.reshape(_B, _S, 3, _NH, _DH).transpose(2, 0, 3, 1, 4)
    qkv = qkv.reshape(3, _B * _NH, _S, _DH)
    o = _attention(qkv[0], qkv[1], qkv[2])
    o = o.reshape(_B, _NH, _S, _DH).transpose(0, 2, 1, 3).reshape(_N, _H)
    x2, y = _proj_res_ln(o, ap['wo'], ap['bo'], h, lp['ln2_g'], lp['ln2_b'])

    c, aux = _router(y, mp['wr'], mp['br'])
    base = _shared(y, mp)

    kstack = jnp.stack([mp['experts'][e]['kernel'].T for e in _CONV_E])
    cv = _convin(y.reshape(_B, _S, _H), kstack).reshape(3, _N, _H)
    yb = y.astype(bf16)
    xl = jnp.stack([cv[0], yb, cv[1], yb, cv[2]])
    w1l = jnp.stack([mp['experts'][e]['w1'] for e in _LOCAL_E]).astype(bf16)
    b1l = jnp.stack([mp['experts'][e]['b1'] for e in _LOCAL_E]).reshape(5, 1, _FF)
    w2l = jnp.stack([mp['experts'][e]['w2'] for e in _LOCAL_E]).astype(bf16)
    b2l = jnp.stack([mp['experts'][e]['b2'] for e in _LOCAL_E]).reshape(5, 1, _H)
    el = _experts_mlp(xl, w1l, b1l, w2l, b2l, _BM)

    yt = y.reshape(_B, _S, _H).transpose(1, 0, 2).reshape(_S, _B * _H).astype(bf16)
    frp = _mm(fwd_c, yt, 256, _B * _H)
    fr = frp[:2 * _F]
    frc = fr.reshape(2, _F, _B, _H).transpose(2, 1, 0, 3).reshape(_B * _F, 2 * _H)
    frc = frc.astype(bf16).reshape(1, _B * _F, 2 * _H)
    w1f = jnp.stack([mp['experts'][e]['w1'] for e in _FOUR_E]).astype(bf16)
    b1f = jnp.stack([mp['experts'][e]['b1'] for e in _FOUR_E]).reshape(3, 1, _FF)
    w2f = jnp.stack([mp['experts'][e]['w2'] for e in _FOUR_E]).astype(bf16)
    b2f = jnp.stack([mp['experts'][e]['b2'] for e in _FOUR_E]).reshape(3, 1, 2 * _H)
    fo = _experts_mlp(frc, w1f, b1f, w2f, b2f, _F, bcast_x=True)

    rhs = fo.reshape(3, _B, _F, 2, _H).transpose(3, 2, 0, 1, 4)
    rhs = rhs.reshape(2 * _F, 3 * _B * _H)
    rhs = jnp.pad(rhs, ((0, _KP - 2 * _F), (0, 0))).astype(bf16)
    t = _mm(inv_c, rhs, 256, 1536)
    ef = t.reshape(_S, 3, _B, _H).transpose(1, 2, 0, 3).reshape(3, _N, _H).astype(bf16)

    h_out = _combine(x2, base, c, el, ef)
    return h_out, aux


def kernel(x, params):
    fwd_c = jnp.asarray(_FWD_NP).astype(bf16)
    inv_c = jnp.asarray(_INV_NP).astype(bf16)
    h = x.reshape(_N, _H)
    aux_total = None
    for lp in params['layers']:
        h, aux = _layer(h, lp, fwd_c, inv_c)
        aux_total = aux if aux_total is None else aux_total + aux
    return h.reshape(_B, _S, _H), aux_total.reshape(())


# speedup vs baseline: 1.3406x; 1.3406x over previous
import numpy as np
import jax
import jax.numpy as jnp
from jax.experimental import pallas as pl
from jax.experimental.pallas import tpu as pltpu

_B, _S, _H, _NH, _E, _TOPK = 2, 2048, 768, 12, 8, 2
_FF = 2 * _H
_DH = _H // _NH
_N = _B * _S
_BM = 256
_F = _S // 2 + 1
_FP = 2304
_KP = 2176

f32 = jnp.float32
bf16 = jnp.bfloat16

_LOCAL_E = (0, 2, 3, 5, 6)
_CONV_E = (0, 3, 6)
_FOUR_E = (1, 4, 7)


def _dft_consts():
    s = np.arange(_S)
    f = np.arange(_F)
    ang = 2.0 * np.pi * np.outer(f, s) / _S
    fwd = np.zeros((_FP, _S), np.float32)
    fwd[:_F] = np.cos(ang)
    fwd[_F:2 * _F] = -np.sin(ang)
    angT = ang.T
    gr = np.cos(angT) * (2.0 / _S)
    gr[:, 0] = 1.0 / _S
    gr[:, _F - 1] = np.cos(np.pi * s) / _S
    gi = -np.sin(angT) * (2.0 / _S)
    gi[:, 0] = 0.0
    gi[:, _F - 1] = 0.0
    inv = np.zeros((_S, _KP), np.float32)
    inv[:, :_F] = gr
    inv[:, _F:2 * _F] = gi
    return fwd, inv


_FWD_NP, _INV_NP = _dft_consts()


def _gelu_exact(x):
    return 0.5 * x * (1.0 + jax.lax.erf(x * 0.7071067811865476))


def _ln_mm_k(x_ref, g_ref, b_ref, w_ref, bb_ref, o_ref):
    x = x_ref[...]
    m = jnp.mean(x, -1, keepdims=True)
    xc = x - m
    v = jnp.mean(xc * xc, -1, keepdims=True)
    z = (xc / jnp.sqrt(v + 1e-5)) * g_ref[...] + b_ref[...]
    z = z.astype(bf16)
    w = w_ref[...].astype(bf16)
    o_ref[...] = jnp.dot(z, w, preferred_element_type=f32) + bb_ref[...]


def _attn_k(q_ref, k_ref, v_ref, o_ref):
    q = q_ref[0].astype(bf16)
    k = k_ref[0].astype(bf16)
    s = jax.lax.dot_general(q, k, (((1,), (1,)), ((), ())),
                            preferred_element_type=f32) * (1.0 / np.sqrt(_DH))
    m = jnp.max(s, -1, keepdims=True)
    p = jnp.exp(s - m)
    p = (p / jnp.sum(p, -1, keepdims=True)).astype(bf16)
    v = v_ref[0].astype(bf16)
    o_ref[0] = jnp.dot(p, v, preferred_element_type=f32)


def _proj_res_ln_k(o_ref, w_ref, bb_ref, xres_ref, g_ref, b_ref, x2_ref, y_ref):
    o = o_ref[...].astype(bf16)
    w = w_ref[...].astype(bf16)
    x2 = jnp.dot(o, w, preferred_element_type=f32) + bb_ref[...] + xres_ref[...]
    x2_ref[...] = x2
    m = jnp.mean(x2, -1, keepdims=True)
    xc = x2 - m
    v = jnp.mean(xc * xc, -1, keepdims=True)
    y_ref[...] = (xc / jnp.sqrt(v + 1e-5)) * g_ref[...] + b_ref[...]


def _router_k(y_ref, wr_ref, br_ref, c_ref, aux_ref, me_ref, cnt_ref):
    i = pl.program_id(0)
    nblk = pl.num_programs(0)
    y = y_ref[...]
    logits = jnp.dot(y.astype(bf16), wr_ref[...].astype(bf16),
                     preferred_element_type=f32) + br_ref[...]
    mx = jnp.max(logits, -1, keepdims=True)
    ex = jnp.exp(logits - mx)
    probs = ex / jnp.sum(ex, -1, keepdims=True)
    iota = jax.lax.broadcasted_iota(jnp.int32, probs.shape, 1)
    m1 = jnp.max(probs, -1, keepdims=True)
    i1 = jnp.min(jnp.where(probs == m1, iota, _E), -1, keepdims=True)
    p2 = jnp.where(iota == i1, -1.0, probs)
    m2 = jnp.max(p2, -1, keepdims=True)
    i2 = jnp.min(jnp.where(p2 == m2, iota, _E), -1, keepdims=True)
    den = m1 + m2
    sel1 = iota == i1
    sel2 = iota == i2
    c_ref[...] = (jnp.where(sel1, m1 / den, 0.0)
                  + jnp.where(sel2, m2 / den, 0.0))

    @pl.when(i == 0)
    def _():
        me_ref[...] = jnp.zeros_like(me_ref)
        cnt_ref[...] = jnp.zeros_like(cnt_ref)

    me_ref[...] += jnp.sum(probs, 0, keepdims=True)
    cnt_ref[...] += jnp.sum(jnp.where(sel1, 1.0, 0.0)
                            + jnp.where(sel2, 1.0, 0.0), 0, keepdims=True)

    @pl.when(i == nblk - 1)
    def _():
        me = me_ref[...] / _N
        ce = cnt_ref[...] / (_N * _TOPK)
        aux_ref[...] = (_E * jnp.sum(me * ce)).reshape(1, 1)


def _shared_k(y_ref, w1_ref, b1_ref, w2_ref, b2_ref, wg_ref, bg_ref, o_ref):
    y = y_ref[...]
    h = jnp.dot(y.astype(bf16), w1_ref[...].astype(bf16),
                preferred_element_type=f32) + b1_ref[...]
    h = _gelu_exact(h)
    s = jnp.dot(h.astype(bf16), w2_ref[...].astype(bf16),
                preferred_element_type=f32) + b2_ref[...]
    g = jnp.sum(y.astype(bf16).astype(f32) * wg_ref[...].astype(bf16).astype(f32),
                -1, keepdims=True) + bg_ref[...]
    o_ref[...] = s * jax.nn.sigmoid(g)


def _convin_k(y_ref, k_ref, o_ref):
    y = y_ref[0]
    z = jnp.zeros((1, y.shape[1]), f32)
    prev = jnp.concatenate([z, y[:-1]], 0)
    nxt = jnp.concatenate([y[1:], z], 0)
    for e in range(3):
        c = prev * k_ref[e, 0:1, :] + y * k_ref[e, 1:2, :] + nxt * k_ref[e, 2:3, :]
        o_ref[e, 0] = y + c


def _emlp_k(x_ref, w1_ref, b1_ref, w2_ref, b2_ref, o_ref):
    x = x_ref[0].astype(bf16)
    h = jnp.dot(x, w1_ref[0].astype(bf16), preferred_element_type=f32) + b1_ref[0]
    h = _gelu_exact(h)
    o_ref[0] = jnp.dot(h.astype(bf16), w2_ref[0].astype(bf16),
                       preferred_element_type=f32) + b2_ref[0]


def _mm_k(a_ref, b_ref, o_ref):
    o_ref[...] = jnp.dot(a_ref[...].astype(bf16), b_ref[...].astype(bf16),
                         preferred_element_type=f32)


def _combine_k(x2_ref, base_ref, c_ref, el_ref, ef_ref, o_ref):
    out = x2_ref[...] + base_ref[...]
    c = c_ref[...]
    for j, e in enumerate(_LOCAL_E):
        out += c[:, e:e + 1] * el_ref[j]
    for j, e in enumerate(_FOUR_E):
        out += c[:, e:e + 1] * ef_ref[j]
    o_ref[...] = out


def _row(x):
    return x.reshape(1, -1)


def _ln_mm(x, g, b, w, bb):
    n, hi = x.shape
    ho = w.shape[1]
    return pl.pallas_call(
        _ln_mm_k,
        grid=(n // _BM,),
        in_specs=[pl.BlockSpec((_BM, hi), lambda i: (i, 0)),
                  pl.BlockSpec((1, hi), lambda i: (0, 0)),
                  pl.BlockSpec((1, hi), lambda i: (0, 0)),
                  pl.BlockSpec((hi, ho), lambda i: (0, 0)),
                  pl.BlockSpec((1, ho), lambda i: (0, 0))],
        out_specs=pl.BlockSpec((_BM, ho), lambda i: (i, 0)),
        out_shape=jax.ShapeDtypeStruct((n, ho), f32),
    )(x, _row(g), _row(b), w, _row(bb))


def _attention(q, k, v):
    gn = q.shape[0]
    bq = 512
    return pl.pallas_call(
        _attn_k,
        grid=(gn, _S // bq),
        in_specs=[pl.BlockSpec((1, bq, _DH), lambda h, i: (h, i, 0)),
                  pl.BlockSpec((1, _S, _DH), lambda h, i: (h, 0, 0)),
                  pl.BlockSpec((1, _S, _DH), lambda h, i: (h, 0, 0))],
        out_specs=pl.BlockSpec((1, bq, _DH), lambda h, i: (h, i, 0)),
        out_shape=jax.ShapeDtypeStruct((gn, _S, _DH), f32),
    )(q, k, v)


def _proj_res_ln(o, w, bb, xres, g, b):
    return pl.pallas_call(
        _proj_res_ln_k,
        grid=(_N // _BM,),
        in_specs=[pl.BlockSpec((_BM, _H), lambda i: (i, 0)),
                  pl.BlockSpec((_H, _H), lambda i: (0, 0)),
                  pl.BlockSpec((1, _H), lambda i: (0, 0)),
                  pl.BlockSpec((_BM, _H), lambda i: (i, 0)),
                  pl.BlockSpec((1, _H), lambda i: (0, 0)),
                  pl.BlockSpec((1, _H), lambda i: (0, 0))],
        out_specs=[pl.BlockSpec((_BM, _H), lambda i: (i, 0)),
                   pl.BlockSpec((_BM, _H), lambda i: (i, 0))],
        out_shape=[jax.ShapeDtypeStruct((_N, _H), f32),
                   jax.ShapeDtypeStruct((_N, _H), f32)],
    )(o, w, _row(bb), xres, _row(g), _row(b))


def _router(y, wr, br):
    c, aux = pl.pallas_call(
        _router_k,
        grid=(_N // _BM,),
        in_specs=[pl.BlockSpec((_BM, _H), lambda i: (i, 0)),
                  pl.BlockSpec((_H, _E), lambda i: (0, 0)),
                  pl.BlockSpec((1, _E), lambda i: (0, 0))],
        out_specs=[pl.BlockSpec((_BM, _E), lambda i: (i, 0)),
                   pl.BlockSpec((1, 1), lambda i: (0, 0))],
        out_shape=[jax.ShapeDtypeStruct((_N, _E), f32),
                   jax.ShapeDtypeStruct((1, 1), f32)],
        scratch_shapes=[pltpu.VMEM((1, _E), f32), pltpu.VMEM((1, _E), f32)],
    )(y, wr, _row(br))
    return c, aux


def _shared(y, mp):
    return pl.pallas_call(
        _shared_k,
        grid=(_N // _BM,),
        in_specs=[pl.BlockSpec((_BM, _H), lambda i: (i, 0)),
                  pl.BlockSpec((_H, _FF), lambda i: (0, 0)),
                  pl.BlockSpec((1, _FF), lambda i: (0, 0)),
                  pl.BlockSpec((_FF, _H), lambda i: (0, 0)),
                  pl.BlockSpec((1, _H), lambda i: (0, 0)),
                  pl.BlockSpec((1, _H), lambda i: (0, 0)),
                  pl.BlockSpec((1, 1), lambda i: (0, 0))],
        out_specs=pl.BlockSpec((_BM, _H), lambda i: (i, 0)),
        out_shape=jax.ShapeDtypeStruct((_N, _H), f32),
    )(y, mp['shared']['w1'], _row(mp['shared']['b1']),
      mp['shared']['w2'], _row(mp['shared']['b2']),
      mp['wg'].reshape(1, _H), mp['bg'].reshape(1, 1))


def _convin(y3, kstack):
    return pl.pallas_call(
        _convin_k,
        grid=(_B, _H // 256),
        in_specs=[pl.BlockSpec((1, _S, 256), lambda b, j: (b, 0, j)),
                  pl.BlockSpec((3, 3, 256), lambda b, j: (0, 0, j))],
        out_specs=pl.BlockSpec((3, 1, _S, 256), lambda b, j: (0, b, 0, j)),
        out_shape=jax.ShapeDtypeStruct((3, _B, _S, _H), f32),
    )(y3, kstack)


def _experts_mlp(xs, w1, b1, w2, b2, bm, bcast_x=False):
    ne = w1.shape[0]
    n, din = xs.shape[1], xs.shape[2]
    dff = w1.shape[2]
    dout = w2.shape[2]
    if bcast_x:
        x_spec = pl.BlockSpec((1, bm, din), lambda e, i: (0, i, 0))
    else:
        x_spec = pl.BlockSpec((1, bm, din), lambda e, i: (e, i, 0))
    return pl.pallas_call(
        _emlp_k,
        grid=(ne, n // bm),
        in_specs=[x_spec,
                  pl.BlockSpec((1, din, dff), lambda e, i: (e, 0, 0)),
                  pl.BlockSpec((1, 1, dff), lambda e, i: (e, 0, 0)),
                  pl.BlockSpec((1, dff, dout), lambda e, i: (e, 0, 0)),
                  pl.BlockSpec((1, 1, dout), lambda e, i: (e, 0, 0))],
        out_specs=pl.BlockSpec((1, bm, dout), lambda e, i: (e, i, 0)),
        out_shape=jax.ShapeDtypeStruct((ne, n, dout), f32),
    )(xs, w1, b1, w2, b2)


def _mm(a, b, bm_a, bn_b):
    ma, ka = a.shape
    kb, nb = b.shape
    return pl.pallas_call(
        _mm_k,
        grid=(ma // bm_a, nb // bn_b),
        in_specs=[pl.BlockSpec((bm_a, ka), lambda i, j: (i, 0)),
                  pl.BlockSpec((kb, bn_b), lambda i, j: (0, j))],
        out_specs=pl.BlockSpec((bm_a, bn_b), lambda i, j: (i, j)),
        out_shape=jax.ShapeDtypeStruct((ma, nb), f32),
    )(a, b)


def _combine(x2, base, c, el, ef):
    return pl.pallas_call(
        _combine_k,
        grid=(_N // _BM,),
        in_specs=[pl.BlockSpec((_BM, _H), lambda i: (i, 0)),
                  pl.BlockSpec((_BM, _H), lambda i: (i, 0)),
                  pl.BlockSpec((_BM, _E), lambda i: (i, 0)),
                  pl.BlockSpec((5, _BM, _H), lambda i: (0, i, 0)),
                  pl.BlockSpec((3, _BM, _H), lambda i: (0, i, 0))],
        out_specs=pl.BlockSpec((_BM, _H), lambda i: (i, 0)),
        out_shape=jax.ShapeDtypeStruct((_N, _H), f32),
    )(x2, base, c, el, ef)


def _layer(h, lp, fwd_c, inv_c):
    ap = lp['attn']
    mp = lp['moe']
    wqkv = jnp.concatenate([ap['wq'], ap['wk'], ap['wv']], axis=1)
    bqkv = jnp.concatenate([ap['bq'], ap['bk'], ap['bv']])
    qkv = _ln_mm(h, lp['ln1_g'], lp['ln1_b'], wqkv, bqkv)
    qkv = qkv.reshape(_B, _S, 3, _NH, _DH).transpose(2, 0, 3, 1, 4)
    qkv = qkv.reshape(3, _B * _NH, _S, _DH)
    o = _attention(qkv[0], qkv[1], qkv[2])
    o = o.reshape(_B, _NH, _S, _DH).transpose(0, 2, 1, 3).reshape(_N, _H)
    x2, y = _proj_res_ln(o, ap['wo'], ap['bo'], h, lp['ln2_g'], lp['ln2_b'])

    c, aux = _router(y, mp['wr'], mp['br'])
    base = _shared(y, mp)

    kstack = jnp.stack([mp['experts'][e]['kernel'].T for e in _CONV_E])
    cv = _convin(y.reshape(_B, _S, _H), kstack).reshape(3, _N, _H)
    xl = jnp.stack([cv[0], y, cv[1], y, cv[2]])
    w1l = jnp.stack([mp['experts'][e]['w1'] for e in _LOCAL_E])
    b1l = jnp.stack([mp['experts'][e]['b1'] for e in _LOCAL_E]).reshape(5, 1, _FF)
    w2l = jnp.stack([mp['experts'][e]['w2'] for e in _LOCAL_E])
    b2l = jnp.stack([mp['experts'][e]['b2'] for e in _LOCAL_E]).reshape(5, 1, _H)
    el = _experts_mlp(xl, w1l, b1l, w2l, b2l, _BM)

    yt = y.reshape(_B, _S, _H).transpose(1, 0, 2).reshape(_S, _B * _H)
    frp = _mm(fwd_c, yt, 256, _B * _H)
    fr = frp[:2 * _F]
    frc = fr.reshape(2, _F, _B, _H).transpose(2, 1, 0, 3).reshape(_B * _F, 2 * _H)
    frc = jnp.pad(frc, ((0, 2560 - 2 * _F), (0, 0))).reshape(1, 2560, 2 * _H)
    w1f = jnp.stack([mp['experts'][e]['w1'] for e in _FOUR_E])
    b1f = jnp.stack([mp['experts'][e]['b1'] for e in _FOUR_E]).reshape(3, 1, _FF)
    w2f = jnp.stack([mp['experts'][e]['w2'] for e in _FOUR_E])
    b2f = jnp.stack([mp['experts'][e]['b2'] for e in _FOUR_E]).reshape(3, 1, 2 * _H)
    fo = _experts_mlp(frc, w1f, b1f, w2f, b2f, 256, bcast_x=True)[:, :2 * _F]

    rhs = fo.reshape(3, _B, _F, 2, _H).transpose(3, 2, 0, 1, 4)
    rhs = rhs.reshape(2 * _F, 3 * _B * _H)
    rhs = jnp.pad(rhs, ((0, _KP - 2 * _F), (0, 0)))
    t = _mm(inv_c, rhs, 256, 1536)
    ef = t.reshape(_S, 3, _B, _H).transpose(1, 2, 0, 3).reshape(3, _N, _H).astype(bf16)

    h_out = _combine(x2, base, c, el, ef)
    return h_out, aux


def kernel(x, params):
    fwd_c = jnp.asarray(_FWD_NP)
    inv_c = jnp.asarray(_INV_NP)
    h = x.reshape(_N, _H)
    aux_total = None
    for lp in params['layers']:
        h, aux = _layer(h, lp, fwd_c, inv_c)
        aux_total = aux if aux_total is None else aux_total + aux
    return h.reshape(_B, _S, _H), aux_total.reshape(())


# speedup vs baseline: 2.7433x; 2.0463x over previous
import numpy as np
import jax
import jax.numpy as jnp
from jax.experimental import pallas as pl
from jax.experimental.pallas import tpu as pltpu

_B, _S, _H, _NH, _E, _TOPK = 2, 2048, 768, 12, 8, 2
_FF = 2 * _H
_DH = _H // _NH
_N = _B * _S
_BM = 256
_F = _S // 2 + 1
_FPAD = 1152

f32 = jnp.float32
bf16 = jnp.bfloat16

_LOCAL_E = (0, 2, 3, 5, 6)
_CONV_E = (0, 3, 6)
_FOUR_E = (1, 4, 7)
_PERM = _LOCAL_E + _FOUR_E


def _dft_consts():
    s = np.arange(_S)
    f = np.arange(_F)
    ang = 2.0 * np.pi * np.outer(f, s) / _S
    fre = np.zeros((_FPAD, _S), np.float32)
    fim = np.zeros((_FPAD, _S), np.float32)
    fre[:_F] = np.cos(ang)
    fim[:_F] = -np.sin(ang)
    angT = ang.T
    gr = np.cos(angT) * (2.0 / _S)
    gr[:, 0] = 1.0 / _S
    gr[:, _F - 1] = np.cos(np.pi * s) / _S
    gi = -np.sin(angT) * (2.0 / _S)
    gi[:, 0] = 0.0
    gi[:, _F - 1] = 0.0
    ginvr = np.zeros((_S, _FPAD), np.float32)
    ginvi = np.zeros((_S, _FPAD), np.float32)
    ginvr[:, :_F] = gr
    ginvi[:, :_F] = gi
    return fre, fim, ginvr, ginvi


_FRE_NP, _FIM_NP, _GR_NP, _GI_NP = _dft_consts()


def _gelu_exact(x):
    return 0.5 * x * (1.0 + jax.lax.erf(x * 0.7071067811865476))


def _ln_mm_k(x_ref, g_ref, b_ref, w_ref, bb_ref, o_ref):
    x = x_ref[...]
    m = jnp.mean(x, -1, keepdims=True)
    xc = x - m
    v = jnp.mean(xc * xc, -1, keepdims=True)
    z = (xc / jnp.sqrt(v + 1e-5)) * g_ref[...] + b_ref[...]
    z = z.astype(bf16)
    w = w_ref[...].astype(bf16)
    o_ref[...] = (jnp.dot(z, w, preferred_element_type=f32)
                  + bb_ref[...]).astype(bf16)


def _attn_k(q_ref, k_ref, v_ref, o_ref):
    for jh in range(2):
        q = q_ref[:, jh * _DH:(jh + 1) * _DH]
        k = k_ref[:, jh * _DH:(jh + 1) * _DH]
        s = jax.lax.dot_general(q, k, (((1,), (1,)), ((), ())),
                                preferred_element_type=f32) * (1.0 / np.sqrt(_DH))
        m = jnp.max(s, -1, keepdims=True)
        p = jnp.exp(s - m)
        p = (p / jnp.sum(p, -1, keepdims=True)).astype(bf16)
        o_ref[:, jh * _DH:(jh + 1) * _DH] = jnp.dot(
            p, v_ref[:, jh * _DH:(jh + 1) * _DH],
            preferred_element_type=f32).astype(bf16)


def _proj_res_ln_k(o_ref, w_ref, bb_ref, xres_ref, g_ref, b_ref, x2_ref, y_ref):
    o = o_ref[...]
    w = w_ref[...].astype(bf16)
    x2 = jnp.dot(o, w, preferred_element_type=f32) + bb_ref[...] + xres_ref[...]
    x2_ref[...] = x2
    m = jnp.mean(x2, -1, keepdims=True)
    xc = x2 - m
    v = jnp.mean(xc * xc, -1, keepdims=True)
    y_ref[...] = (xc / jnp.sqrt(v + 1e-5)) * g_ref[...] + b_ref[...]


def _router_k(y_ref, wr_ref, br_ref, c_ref, aux_ref, me_ref, cnt_ref):
    i = pl.program_id(0)
    nblk = pl.num_programs(0)
    y = y_ref[...]
    logits = jnp.dot(y.astype(bf16), wr_ref[...].astype(bf16),
                     preferred_element_type=f32) + br_ref[...]
    mx = jnp.max(logits, -1, keepdims=True)
    ex = jnp.exp(logits - mx)
    probs = ex / jnp.sum(ex, -1, keepdims=True)
    iota = jax.lax.broadcasted_iota(jnp.int32, probs.shape, 1)
    m1 = jnp.max(probs, -1, keepdims=True)
    i1 = jnp.min(jnp.where(probs == m1, iota, _E), -1, keepdims=True)
    p2 = jnp.where(iota == i1, -1.0, probs)
    m2 = jnp.max(p2, -1, keepdims=True)
    i2 = jnp.min(jnp.where(p2 == m2, iota, _E), -1, keepdims=True)
    den = m1 + m2
    sel1 = iota == i1
    sel2 = iota == i2
    c = jnp.where(sel1, m1 / den, 0.0) + jnp.where(sel2, m2 / den, 0.0)
    c_ref[...] = jnp.concatenate([c[:, e:e + 1] for e in _PERM], axis=1)

    @pl.when(i == 0)
    def _():
        me_ref[...] = jnp.zeros_like(me_ref)
        cnt_ref[...] = jnp.zeros_like(cnt_ref)

    me_ref[...] += jnp.sum(probs, 0, keepdims=True)
    cnt_ref[...] += jnp.sum(jnp.where(sel1, 1.0, 0.0)
                            + jnp.where(sel2, 1.0, 0.0), 0, keepdims=True)

    @pl.when(i == nblk - 1)
    def _():
        me = me_ref[...] / _N
        ce = cnt_ref[...] / (_N * _TOPK)
        aux_ref[...] = (_E * jnp.sum(me * ce)).reshape(1, 1)


def _shared_k(y_ref, w1_ref, b1_ref, w2_ref, b2_ref, wg_ref, bg_ref, o_ref):
    y = y_ref[...]
    h = jnp.dot(y.astype(bf16), w1_ref[...].astype(bf16),
                preferred_element_type=f32) + b1_ref[...]
    h = _gelu_exact(h)
    s = jnp.dot(h.astype(bf16), w2_ref[...].astype(bf16),
                preferred_element_type=f32) + b2_ref[...]
    g = jnp.sum(y.astype(bf16).astype(f32) * wg_ref[...].astype(bf16).astype(f32),
                -1, keepdims=True) + bg_ref[...]
    o_ref[...] = s * jax.nn.sigmoid(g)


def _convin_k(y_ref, k_ref, o_ref):
    y = y_ref[0]
    z = jnp.zeros((1, y.shape[1]), f32)
    prev = jnp.concatenate([z, y[:-1]], 0)
    nxt = jnp.concatenate([y[1:], z], 0)
    for e in range(3):
        c = prev * k_ref[e, 0:1, :] + y * k_ref[e, 1:2, :] + nxt * k_ref[e, 2:3, :]
        o_ref[e, 0] = y + c


def _expcomb_k(cv_ref, y_ref, w1_ref, b1_ref, w2_ref, b2_ref,
               c_ref, x2_ref, base_ref, t0_ref, t1_ref, t2_ref, o_ref):
    e = pl.program_id(1)
    x = jnp.where((e % 2) == 0, cv_ref[0], y_ref[...]).astype(bf16)
    h = jnp.dot(x, w1_ref[0], preferred_element_type=f32) + b1_ref[0]
    h = _gelu_exact(h)
    oe = jnp.dot(h.astype(bf16), w2_ref[0], preferred_element_type=f32) + b2_ref[0]
    call = c_ref[...]
    iota = jax.lax.broadcasted_iota(jnp.int32, call.shape, 1)
    coef = jnp.sum(jnp.where(iota == e, call, 0.0), axis=1, keepdims=True)

    @pl.when(e == 0)
    def _():
        o_ref[...] = x2_ref[...] + base_ref[...] + coef * oe

    @pl.when(e > 0)
    def _():
        o_ref[...] += coef * oe

    @pl.when(e == 4)
    def _():
        acc = o_ref[...]
        for j, t_ref in enumerate((t0_ref, t1_ref, t2_ref)):
            acc += call[:, 5 + j:6 + j] * t_ref[0, 0]
        o_ref[...] = acc


def _fmlp_k(re_ref, im_ref, w1_ref, b1_ref, w2_ref, b2_ref, o_ref):
    x = jnp.concatenate([re_ref[...], im_ref[...]], axis=1).astype(bf16)
    h = jnp.dot(x, w1_ref[0], preferred_element_type=f32) + b1_ref[0]
    h = _gelu_exact(h)
    o_ref[0, 0] = (jnp.dot(h.astype(bf16), w2_ref[0],
                           preferred_element_type=f32) + b2_ref[0]).astype(bf16)


def _idft_k(gr_ref, gi_ref, re_ref, im_ref, o_ref):
    o = jnp.dot(gr_ref[...].astype(bf16), re_ref[0, 0],
                preferred_element_type=f32)
    o += jnp.dot(gi_ref[...].astype(bf16), im_ref[0, 0],
                 preferred_element_type=f32)
    o_ref[0, 0] = o


def _mm_k(a_ref, b_ref, o_ref):
    o_ref[...] = jnp.dot(a_ref[...].astype(bf16), b_ref[...].astype(bf16),
                         preferred_element_type=f32)


def _row(x):
    return x.reshape(1, -1)


def _ln_mm(x, g, b, w, bb):
    n, hi = x.shape
    ho = w.shape[1]
    return pl.pallas_call(
        _ln_mm_k,
        grid=(n // _BM,),
        in_specs=[pl.BlockSpec((_BM, hi), lambda i: (i, 0)),
                  pl.BlockSpec((1, hi), lambda i: (0, 0)),
                  pl.BlockSpec((1, hi), lambda i: (0, 0)),
                  pl.BlockSpec((hi, ho), lambda i: (0, 0)),
                  pl.BlockSpec((1, ho), lambda i: (0, 0))],
        out_specs=pl.BlockSpec((_BM, ho), lambda i: (i, 0)),
        out_shape=jax.ShapeDtypeStruct((n, ho), bf16),
    )(x, _row(g), _row(b), w, _row(bb))


def _attention(qkv):
    bq = 512
    nsb = _S // bq
    return pl.pallas_call(
        _attn_k,
        grid=(_N // bq, _NH // 2),
        in_specs=[pl.BlockSpec((bq, 128), lambda i, j: (i, j)),
                  pl.BlockSpec((_S, 128), lambda i, j: (i // nsb, 6 + j)),
                  pl.BlockSpec((_S, 128), lambda i, j: (i // nsb, 12 + j))],
        out_specs=pl.BlockSpec((bq, 128), lambda i, j: (i, j)),
        out_shape=jax.ShapeDtypeStruct((_N, _H), bf16),
    )(qkv, qkv, qkv)


def _proj_res_ln(o, w, bb, xres, g, b):
    return pl.pallas_call(
        _proj_res_ln_k,
        grid=(_N // _BM,),
        in_specs=[pl.BlockSpec((_BM, _H), lambda i: (i, 0)),
                  pl.BlockSpec((_H, _H), lambda i: (0, 0)),
                  pl.BlockSpec((1, _H), lambda i: (0, 0)),
                  pl.BlockSpec((_BM, _H), lambda i: (i, 0)),
                  pl.BlockSpec((1, _H), lambda i: (0, 0)),
                  pl.BlockSpec((1, _H), lambda i: (0, 0))],
        out_specs=[pl.BlockSpec((_BM, _H), lambda i: (i, 0)),
                   pl.BlockSpec((_BM, _H), lambda i: (i, 0))],
        out_shape=[jax.ShapeDtypeStruct((_N, _H), f32),
                   jax.ShapeDtypeStruct((_N, _H), f32)],
    )(o, w, _row(bb), xres, _row(g), _row(b))


def _router(y, wr, br):
    c, aux = pl.pallas_call(
        _router_k,
        grid=(_N // _BM,),
        in_specs=[pl.BlockSpec((_BM, _H), lambda i: (i, 0)),
                  pl.BlockSpec((_H, _E), lambda i: (0, 0)),
                  pl.BlockSpec((1, _E), lambda i: (0, 0))],
        out_specs=[pl.BlockSpec((_BM, _E), lambda i: (i, 0)),
                   pl.BlockSpec((1, 1), lambda i: (0, 0))],
        out_shape=[jax.ShapeDtypeStruct((_N, _E), f32),
                   jax.ShapeDtypeStruct((1, 1), f32)],
        scratch_shapes=[pltpu.VMEM((1, _E), f32), pltpu.VMEM((1, _E), f32)],
    )(y, wr, _row(br))
    return c, aux


def _shared(y, mp):
    return pl.pallas_call(
        _shared_k,
        grid=(_N // _BM,),
        in_specs=[pl.BlockSpec((_BM, _H), lambda i: (i, 0)),
                  pl.BlockSpec((_H, _FF), lambda i: (0, 0)),
                  pl.BlockSpec((1, _FF), lambda i: (0, 0)),
                  pl.BlockSpec((_FF, _H), lambda i: (0, 0)),
                  pl.BlockSpec((1, _H), lambda i: (0, 0)),
                  pl.BlockSpec((1, _H), lambda i: (0, 0)),
                  pl.BlockSpec((1, 1), lambda i: (0, 0))],
        out_specs=pl.BlockSpec((_BM, _H), lambda i: (i, 0)),
        out_shape=jax.ShapeDtypeStruct((_N, _H), f32),
    )(y, mp['shared']['w1'], _row(mp['shared']['b1']),
      mp['shared']['w2'], _row(mp['shared']['b2']),
      mp['wg'].reshape(1, _H), mp['bg'].reshape(1, 1))


def _convin(y3, kstack):
    return pl.pallas_call(
        _convin_k,
        grid=(_B, _H // 256),
        in_specs=[pl.BlockSpec((1, _S, 256), lambda b, j: (b, 0, j)),
                  pl.BlockSpec((3, 3, 256), lambda b, j: (0, 0, j))],
        out_specs=pl.BlockSpec((3, 1, _S, 256), lambda b, j: (0, b, 0, j)),
        out_shape=jax.ShapeDtypeStruct((3, _B, _S, _H), f32),
    )(y3, kstack)


def _experts_combine(cv, y, w1l, b1l, w2l, b2l, c, x2, base, t):
    nsb = _S // _BM
    return pl.pallas_call(
        _expcomb_k,
        grid=(_N // _BM, 5),
        in_specs=[pl.BlockSpec((1, _BM, _H), lambda i, e: (e // 2, i, 0)),
                  pl.BlockSpec((_BM, _H), lambda i, e: (i, 0)),
                  pl.BlockSpec((1, _H, _FF), lambda i, e: (e, 0, 0)),
                  pl.BlockSpec((1, 1, _FF), lambda i, e: (e, 0, 0)),
                  pl.BlockSpec((1, _FF, _H), lambda i, e: (e, 0, 0)),
                  pl.BlockSpec((1, 1, _H), lambda i, e: (e, 0, 0)),
                  pl.BlockSpec((_BM, _E), lambda i, e: (i, 0)),
                  pl.BlockSpec((_BM, _H), lambda i, e: (i, 0)),
                  pl.BlockSpec((_BM, _H), lambda i, e: (i, 0)),
                  pl.BlockSpec((1, 1, _BM, _H), lambda i, e: (0, i // nsb, i % nsb, 0)),
                  pl.BlockSpec((1, 1, _BM, _H), lambda i, e: (1, i // nsb, i % nsb, 0)),
                  pl.BlockSpec((1, 1, _BM, _H), lambda i, e: (2, i // nsb, i % nsb, 0))],
        out_specs=pl.BlockSpec((_BM, _H), lambda i, e: (i, 0)),
        out_shape=jax.ShapeDtypeStruct((_N, _H), f32),
    )(cv, y, w1l, b1l, w2l, b2l, c, x2, base, t, t, t)


def _mm(a, b, bm_a, bn_b):
    ma, ka = a.shape
    kb, nb = b.shape
    return pl.pallas_call(
        _mm_k,
        grid=(ma // bm_a, nb // bn_b),
        in_specs=[pl.BlockSpec((bm_a, ka), lambda i, j: (i, 0)),
                  pl.BlockSpec((kb, bn_b), lambda i, j: (0, j))],
        out_specs=pl.BlockSpec((bm_a, bn_b), lambda i, j: (i, j)),
        out_shape=jax.ShapeDtypeStruct((ma, nb), f32),
    )(a, b)


def _fourier_mlp(re, im, w1f, b1f, w2f, b2f):
    bm = 128
    return pl.pallas_call(
        _fmlp_k,
        grid=(3, _B, _FPAD // bm),
        in_specs=[pl.BlockSpec((bm, _H), lambda e, b, i: (i, b)),
                  pl.BlockSpec((bm, _H), lambda e, b, i: (i, b)),
                  pl.BlockSpec((1, 2 * _H, _FF), lambda e, b, i: (e, 0, 0)),
                  pl.BlockSpec((1, 1, _FF), lambda e, b, i: (e, 0, 0)),
                  pl.BlockSpec((1, _FF, 2 * _H), lambda e, b, i: (e, 0, 0)),
                  pl.BlockSpec((1, 1, 2 * _H), lambda e, b, i: (e, 0, 0))],
        out_specs=pl.BlockSpec((1, 1, bm, 2 * _H), lambda e, b, i: (e, b, i, 0)),
        out_shape=jax.ShapeDtypeStruct((3, _B, _FPAD, 2 * _H), bf16),
    )(re, im, w1f, b1f, w2f, b2f)


def _idft(gr, gi, fo):
    bm = 256
    return pl.pallas_call(
        _idft_k,
        grid=(_S // bm, 3, _B),
        in_specs=[pl.BlockSpec((bm, _FPAD), lambda i, e, b: (i, 0)),
                  pl.BlockSpec((bm, _FPAD), lambda i, e, b: (i, 0)),
                  pl.BlockSpec((1, 1, _FPAD, _H), lambda i, e, b: (e, b, 0, 0)),
                  pl.BlockSpec((1, 1, _FPAD, _H), lambda i, e, b: (e, b, 0, 1))],
        out_specs=pl.BlockSpec((1, 1, bm, _H), lambda i, e, b: (e, b, i, 0)),
        out_shape=jax.ShapeDtypeStruct((3, _B, _S, _H), f32),
    )(gr, gi, fo, fo)


def _layer(h, lp, consts):
    fre_c, fim_c, gr_c, gi_c = consts
    ap = lp['attn']
    mp = lp['moe']
    wqkv = jnp.concatenate([ap['wq'], ap['wk'], ap['wv']], axis=1)
    bqkv = jnp.concatenate([ap['bq'], ap['bk'], ap['bv']])
    qkv = _ln_mm(h, lp['ln1_g'], lp['ln1_b'], wqkv, bqkv)
    o = _attention(qkv)
    x2, y = _proj_res_ln(o, ap['wo'], ap['bo'], h,
                         lp['ln2_g'], lp['ln2_b'])

    c, aux = _router(y, mp['wr'], mp['br'])
    base = _shared(y, mp)

    kstack = jnp.stack([mp['experts'][e]['kernel'].T for e in _CONV_E])
    cv = _convin(y.reshape(_B, _S, _H), kstack).reshape(3, _N, _H)

    yt = y.reshape(_B, _S, _H).transpose(1, 0, 2).reshape(_S, _B * _H)
    re = _mm(fre_c, yt, 128, _B * _H)
    im = _mm(fim_c, yt, 128, _B * _H)
    w1f = jnp.stack([mp['experts'][e]['w1'] for e in _FOUR_E]).astype(bf16)
    b1f = jnp.stack([mp['experts'][e]['b1'] for e in _FOUR_E]).reshape(3, 1, _FF)
    w2f = jnp.stack([mp['experts'][e]['w2'] for e in _FOUR_E]).astype(bf16)
    b2f = jnp.stack([mp['experts'][e]['b2'] for e in _FOUR_E]).reshape(3, 1, 2 * _H)
    fo = _fourier_mlp(re, im, w1f, b1f, w2f, b2f)
    t = _idft(gr_c, gi_c, fo)

    w1l = jnp.stack([mp['experts'][e]['w1'] for e in _LOCAL_E]).astype(bf16)
    b1l = jnp.stack([mp['experts'][e]['b1'] for e in _LOCAL_E]).reshape(5, 1, _FF)
    w2l = jnp.stack([mp['experts'][e]['w2'] for e in _LOCAL_E]).astype(bf16)
    b2l = jnp.stack([mp['experts'][e]['b2'] for e in _LOCAL_E]).reshape(5, 1, _H)
    h_out = _experts_combine(cv, y, w1l, b1l, w2l, b2l, c, x2, base, t)
    return h_out, aux


def kernel(x, params):
    consts = (jnp.asarray(_FRE_NP), jnp.asarray(_FIM_NP),
              jnp.asarray(_GR_NP), jnp.asarray(_GI_NP))
    h = x.reshape(_N, _H)
    aux_total = None
    for lp in params['layers']:
        h, aux = _layer(h, lp, consts)
        aux_total = aux if aux_total is None else aux_total + aux
    return h.reshape(_B, _S, _H), aux_total.reshape(())
